# baseline (speedup 1.0000x reference)
import functools

import jax
import jax.numpy as jnp
from jax import lax
from jax.experimental import pallas as pl
from jax.experimental.pallas import tpu as pltpu
from jax.experimental.pallas import tpu_sc as plsc

N = 16384
F = 4
NP = N // F
C = 64
NPOINT = 1024
RADIUS2 = 0.2 * 0.2
NSAMPLE = 32
H1 = 64
H2 = 128
D = 128
BN_EPS = 1e-5

_NW = 32
_CHUNK = 64


_HNP = NP // 2


def _fps_body(x_ref, y_ref, z_ref, sel_ref, qx_ref, qy_ref, qz_ref):
    x = x_ref[...]
    y = y_ref[...]
    z = z_ref[...]
    lane = lax.broadcasted_iota(jnp.int32, (2 * F, _HNP), 1)
    half = lax.broadcasted_iota(jnp.int32, (2 * F, _HNP), 0) // F
    lane = lane + half * _HNP
    qlane = lax.broadcasted_iota(jnp.int32, (F, NPOINT), 1)

    def framered(t, op):
        return op(t[:F], t[F:])

    px0 = x[:F, 0:1]
    py0 = y[:F, 0:1]
    pz0 = z[:F, 0:1]

    def dist2(px, py, pz):
        pxr = jnp.concatenate([px, px], axis=0)
        pyr = jnp.concatenate([py, py], axis=0)
        pzr = jnp.concatenate([pz, pz], axis=0)
        dx = x - pxr
        dy = y - pyr
        dz = z - pzr
        return dx * dx + dy * dy + dz * dz

    mind = dist2(px0, py0, pz0)

    sel0 = jnp.zeros((F, NPOINT), jnp.int32)
    qx0 = jnp.zeros((F, NPOINT), jnp.float32) + px0
    qy0 = jnp.zeros((F, NPOINT), jnp.float32) + py0
    qz0 = jnp.zeros((F, NPOINT), jnp.float32) + pz0

    def body(i, carry):
        mind, sel, qx, qy, qz = carry
        ml = jnp.max(mind, axis=1, keepdims=True)
        mf = framered(ml, jnp.maximum)
        m2 = jnp.concatenate([mf, mf], axis=0)
        il = jnp.min(jnp.where(mind == m2, lane, NP), axis=1, keepdims=True)
        idx = framered(il, jnp.minimum)
        i2 = jnp.concatenate([idx, idx], axis=0)
        eq = lane == i2
        sx = jnp.sum(jnp.where(eq, x, 0.0), axis=1, keepdims=True)
        sy = jnp.sum(jnp.where(eq, y, 0.0), axis=1, keepdims=True)
        sz = jnp.sum(jnp.where(eq, z, 0.0), axis=1, keepdims=True)
        px = framered(sx, jnp.add)
        py = framered(sy, jnp.add)
        pz = framered(sz, jnp.add)
        d = dist2(px, py, pz)
        here = qlane == i
        sel = jnp.where(here, idx, sel)
        qx = jnp.where(here, px, qx)
        qy = jnp.where(here, py, qy)
        qz = jnp.where(here, pz, qz)
        return jnp.minimum(mind, d), sel, qx, qy, qz

    _, sel, qx, qy, qz = lax.fori_loop(
        1, NPOINT, body, (mind, sel0, qx0, qy0, qz0))
    sel_ref[...] = sel
    qx_ref[...] = qx
    qy_ref[...] = qy
    qz_ref[...] = qz


def _fps_call(x, y, z):
    tohalf = lambda a: a.reshape(F, 2, _HNP).transpose(1, 0, 2).reshape(2 * F, _HNP)
    return pl.pallas_call(
        _fps_body,
        out_shape=(
            jax.ShapeDtypeStruct((F, NPOINT), jnp.int32),
            jax.ShapeDtypeStruct((F, NPOINT), jnp.float32),
            jax.ShapeDtypeStruct((F, NPOINT), jnp.float32),
            jax.ShapeDtypeStruct((F, NPOINT), jnp.float32),
        ),
    )(tohalf(x), tohalf(y), tohalf(z))


_QBLK = 256


def _bq_body(x_ref, y_ref, z_ref, sel_ref, qx_ref, qy_ref, qz_ref,
             nbr_ref, val_ref):
    f = pl.program_id(0)
    x = x_ref[0]
    y = y_ref[0]
    z = z_ref[0]
    lane = lax.broadcasted_iota(jnp.int32, (_QBLK, NP), 1)
    slot = lax.broadcasted_iota(jnp.int32, (_QBLK, NSAMPLE), 1)

    for c in range(NPOINT // _QBLK):
        sl = pl.ds(c * _QBLK, _QBLK)
        qx = qx_ref[0, 0, sl][:, None]
        qy = qy_ref[0, 0, sl][:, None]
        qz = qz_ref[0, 0, sl][:, None]
        qloc = sel_ref[0, 0, sl][:, None]
        dx = qx - x
        dy = qy - y
        dz = qz - z
        d2 = dx * dx + dy * dy + dz * dz
        mask = d2 <= RADIUS2
        cnt = jnp.sum(mask.astype(jnp.int32), axis=1, keepdims=True)
        key0 = jnp.where(mask, lane, NP)
        m = jnp.min(key0, axis=1, keepdims=True)
        cols = [m]
        for _ in range(NSAMPLE - 1):
            m = jnp.min(jnp.where(lane > m, key0, NP), axis=1, keepdims=True)
            cols.append(m)
        nbrs = jnp.concatenate(cols, axis=1)
        valid = slot < cnt
        nbr = jnp.where(valid, nbrs, qloc) + f * NP
        nbr_ref[0, sl, :] = nbr
        val_ref[0, sl, :] = valid.astype(jnp.int32)


def _bq_call(x, y, z, sel, qx, qy, qz):
    frame_spec = pl.BlockSpec((1, 1, NP), lambda f: (f, 0, 0))
    q_spec = pl.BlockSpec((1, 1, NPOINT), lambda f: (f, 0, 0))
    out_spec = pl.BlockSpec((1, NPOINT, NSAMPLE), lambda f: (f, 0, 0))
    r3 = lambda a: a.reshape(F, 1, a.shape[-1])
    return pl.pallas_call(
        _bq_body,
        grid=(F,),
        in_specs=[frame_spec, frame_spec, frame_spec,
                  q_spec, q_spec, q_spec, q_spec],
        out_specs=(out_spec, out_spec),
        out_shape=(
            jax.ShapeDtypeStruct((F, NPOINT, NSAMPLE), jnp.int32),
            jax.ShapeDtypeStruct((F, NPOINT, NSAMPLE), jnp.int32),
        ),
    )(r3(x), r3(y), r3(z), r3(sel), r3(qx), r3(qy), r3(qz))


_GB = 4
_NGRP = 16


def _sc_gather(table, nbr_idx, q_idx):
    qns = nbr_idx.shape[0] * nbr_idx.shape[1]
    nq = q_idx.shape[0] * q_idx.shape[1]
    per_w = qns // _NW
    n_chunks = per_w // _CHUNK
    mesh = plsc.VectorSubcoreMesh(core_axis_name="c", subcore_axis_name="s")

    @functools.partial(
        pl.kernel,
        mesh=mesh,
        out_type=[
            jax.ShapeDtypeStruct((qns, D), jnp.float32),
            jax.ShapeDtypeStruct((nq, D), jnp.float32),
        ],
        scratch_types=[
            pltpu.VMEM((n_chunks, _CHUNK), jnp.int32),
            pltpu.VMEM((2, _CHUNK), jnp.int32),
            pltpu.VMEM((2 * _GB, _CHUNK, D), jnp.float32),
            pltpu.SemaphoreType.DMA,
            pltpu.SemaphoreType.DMA,
        ],
    )
    def k(table_hbm, nbr_hbm, q_hbm, outn_hbm, outq_hbm,
          idx_all, qidx, bufs, gsem, wsem):
        wid = lax.axis_index("s") * 2 + lax.axis_index("c")
        pltpu.sync_copy(nbr_hbm.at[pl.ds(wid * n_chunks, n_chunks)], idx_all)
        pltpu.sync_copy(q_hbm.at[pl.ds(wid * 2, 2)], qidx)
        out_base = wid * per_w

        def chunk_out(c):
            off = pl.multiple_of(out_base + c * _CHUNK, _CHUNK)
            return outn_hbm.at[pl.ds(off, _CHUNK)]

        def drain(sem):
            pltpu.make_async_copy(
                table_hbm.at[pl.ds(0, _CHUNK)], bufs.at[0], sem).wait()

        for b in range(2 * _GB):
            pltpu.async_copy(bufs.at[b], chunk_out(b), wsem)

        def group(g, carry):
            p = (g % 2) * _GB
            for b in range(_GB):
                drain(wsem)
            for b in range(_GB):
                c = g * _GB + b
                pltpu.async_copy(table_hbm.at[idx_all.at[c]],
                                 bufs.at[p + b], gsem)
            for b in range(_GB):
                drain(gsem)
            for b in range(_GB):
                c = g * _GB + b
                pltpu.async_copy(bufs.at[p + b], chunk_out(c), wsem)
            return carry

        lax.fori_loop(0, _NGRP, group, 0)
        for b in range(2 * _GB):
            drain(wsem)

        for t in range(2):
            pltpu.async_copy(table_hbm.at[qidx.at[t]], bufs.at[t], gsem)
        for t in range(2):
            drain(gsem)
        for t in range(2):
            off = pl.multiple_of(wid * 2 * _CHUNK + t * _CHUNK, _CHUNK)
            pltpu.async_copy(bufs.at[t], outq_hbm.at[pl.ds(off, _CHUNK)], wsem)
        for t in range(2):
            drain(wsem)

    return k(table, nbr_idx, q_idx)


_MQBLK = 256


def _mlp_body(g_ref, q_ref, val_ref, w1_ref, b1_ref, s1_ref, be1_ref,
              w2_ref, b2_ref, s2_ref, be2_ref, o_ref):
    g = g_ref[...]
    q = q_ref[...]
    col = lax.broadcasted_iota(jnp.int32, (_MQBLK, D), 1)
    qxyz = jnp.where(col < 3, q, 0.0)
    g3 = g.reshape(_MQBLK, NSAMPLE, D) - qxyz[:, None, :]
    a = g3.reshape(_MQBLK * NSAMPLE, D)
    y1 = jax.lax.dot_general(a, w1_ref[...], (((1,), (0,)), ((), ())),
                             preferred_element_type=jnp.float32)
    y1 = (y1 + b1_ref[...]) * s1_ref[...] + be1_ref[...]
    h1 = jnp.maximum(y1, 0.0)
    y2 = jax.lax.dot_general(h1, w2_ref[...], (((1,), (0,)), ((), ())),
                             preferred_element_type=jnp.float32)
    y2 = (y2 + b2_ref[...]) * s2_ref[...] + be2_ref[...]
    h2 = jnp.maximum(y2, 0.0)
    vrow = val_ref[...]
    hm = jnp.where(vrow > 0.5, h2, -jnp.inf)
    o_ref[...] = jnp.max(hm.reshape(_MQBLK, NSAMPLE, H2), axis=1).T


def _mlp_call(gn, gq, valid, w1t, b1, s1, be1, w2t, b2, s2, be2):
    nq = gq.shape[0]
    grid = (nq // _MQBLK,)
    full = lambda shape: pl.BlockSpec(shape, lambda i: tuple(0 for _ in shape))
    return pl.pallas_call(
        _mlp_body,
        grid=grid,
        in_specs=[
            pl.BlockSpec((_MQBLK * NSAMPLE, D), lambda i: (i, 0)),
            pl.BlockSpec((_MQBLK, D), lambda i: (i, 0)),
            pl.BlockSpec((_MQBLK * NSAMPLE, 1), lambda i: (i, 0)),
            full((D, H1)), full((1, H1)), full((1, H1)), full((1, H1)),
            full((H1, H2)), full((1, H2)), full((1, H2)), full((1, H2)),
        ],
        out_specs=pl.BlockSpec((H2, _MQBLK), lambda i: (0, i)),
        out_shape=jax.ShapeDtypeStruct((H2, nq), jnp.float32),
    )(gn, gq, valid, w1t, b1, s1, be1, w2t, b2, s2, be2)


def kernel(xyz, times, features, point2frameidx, frame2batchidx,
           W1, b1, g1, be1, W2, b2, g2, be2):
    p = xyz.reshape(F, NP, 3)
    x = p[:, :, 0]
    y = p[:, :, 1]
    z = p[:, :, 2]

    sel, qx, qy, qz = _fps_call(x, y, z)
    nbr, valid = _bq_call(x, y, z, sel, qx, qy, qz)

    offsets = (jnp.arange(F, dtype=jnp.int32) * NP)[:, None]
    inds = (sel + offsets).reshape(-1)

    pad = jnp.zeros((N, D - 3 - C), jnp.float32)
    table = jnp.concatenate([xyz, features, pad], axis=1)

    gn, gq = _sc_gather(table, nbr.reshape(-1, _CHUNK),
                        inds.reshape(-1, _CHUNK))

    s1 = (g1 / jnp.sqrt(1.0 + BN_EPS))[None, :]
    s2 = (g2 / jnp.sqrt(1.0 + BN_EPS))[None, :]
    w1t = jnp.zeros((D, H1), jnp.float32).at[:3 + C, :].set(W1.T)
    w2t = W2.T
    out = _mlp_call(gn, gq, valid.reshape(-1, 1).astype(jnp.float32),
                    w1t, b1[None, :], s1, be1[None, :],
                    w2t, b2[None, :], s2, be2[None, :])

    query_xyz = gq[:, :3][None]
    new_features = out[None]
    return query_xyz, new_features, inds


# speedup vs baseline: 1.1574x; 1.1574x over previous
import functools

import jax
import jax.numpy as jnp
from jax import lax
from jax.experimental import pallas as pl
from jax.experimental.pallas import tpu as pltpu
from jax.experimental.pallas import tpu_sc as plsc

N = 16384
F = 4
NP = N // F
C = 64
NPOINT = 1024
RADIUS2 = 0.2 * 0.2
NSAMPLE = 32
H1 = 64
H2 = 128
D = 128
BN_EPS = 1e-5

_NW = 32
_CHUNK = 64


def _fps_body(x_ref, y_ref, z_ref, sel_ref, qx_ref, qy_ref, qz_ref):
    x = x_ref[...]
    y = y_ref[...]
    z = z_ref[...]
    lane = lax.broadcasted_iota(jnp.int32, (F, NP), 1)
    qlane = lax.broadcasted_iota(jnp.int32, (F, NPOINT), 1)

    px = x[:, 0:1]
    py = y[:, 0:1]
    pz = z[:, 0:1]
    dx = x - px
    dy = y - py
    dz = z - pz
    mind = dx * dx + dy * dy + dz * dz

    sel0 = jnp.zeros((F, NPOINT), jnp.int32)
    qx0 = jnp.zeros((F, NPOINT), jnp.float32) + px
    qy0 = jnp.zeros((F, NPOINT), jnp.float32) + py
    qz0 = jnp.zeros((F, NPOINT), jnp.float32) + pz

    def body(i, carry):
        mind, sel, qx, qy, qz = carry
        m = jnp.max(mind, axis=1, keepdims=True)
        idx = jnp.min(jnp.where(mind == m, lane, NP), axis=1, keepdims=True)
        eq = lane == idx
        px = jnp.sum(jnp.where(eq, x, 0.0), axis=1, keepdims=True)
        py = jnp.sum(jnp.where(eq, y, 0.0), axis=1, keepdims=True)
        pz = jnp.sum(jnp.where(eq, z, 0.0), axis=1, keepdims=True)
        dx = x - px
        dy = y - py
        dz = z - pz
        d = dx * dx + dy * dy + dz * dz
        here = qlane == i
        sel = jnp.where(here, idx, sel)
        qx = jnp.where(here, px, qx)
        qy = jnp.where(here, py, qy)
        qz = jnp.where(here, pz, qz)
        return jnp.minimum(mind, d), sel, qx, qy, qz

    _, sel, qx, qy, qz = lax.fori_loop(
        1, NPOINT, body, (mind, sel0, qx0, qy0, qz0))
    sel_ref[...] = sel
    qx_ref[...] = qx
    qy_ref[...] = qy
    qz_ref[...] = qz


def _fps_call(x, y, z):
    return pl.pallas_call(
        _fps_body,
        out_shape=(
            jax.ShapeDtypeStruct((F, NPOINT), jnp.int32),
            jax.ShapeDtypeStruct((F, NPOINT), jnp.float32),
            jax.ShapeDtypeStruct((F, NPOINT), jnp.float32),
            jax.ShapeDtypeStruct((F, NPOINT), jnp.float32),
        ),
    )(x, y, z)


_QBLK = 256


def _bq_body(x_ref, y_ref, z_ref, sel_ref, qx_ref, qy_ref, qz_ref,
             nbr_ref, val_ref):
    f = pl.program_id(0)
    x = x_ref[0]
    y = y_ref[0]
    z = z_ref[0]
    lane = lax.broadcasted_iota(jnp.int32, (_QBLK, NP), 1)
    slot = lax.broadcasted_iota(jnp.int32, (_QBLK, NSAMPLE), 1)

    for c in range(NPOINT // _QBLK):
        sl = pl.ds(c * _QBLK, _QBLK)
        qx = qx_ref[0, 0, sl][:, None]
        qy = qy_ref[0, 0, sl][:, None]
        qz = qz_ref[0, 0, sl][:, None]
        qloc = sel_ref[0, 0, sl][:, None]
        dx = qx - x
        dy = qy - y
        dz = qz - z
        d2 = dx * dx + dy * dy + dz * dz
        mask = d2 <= RADIUS2
        cnt = jnp.sum(mask.astype(jnp.int32), axis=1, keepdims=True)
        key0 = jnp.where(mask, lane, NP)
        m = jnp.min(key0, axis=1, keepdims=True)
        cols = [m]
        for _ in range(NSAMPLE - 1):
            m = jnp.min(jnp.where(lane > m, key0, NP), axis=1, keepdims=True)
            cols.append(m)
        nbrs = jnp.concatenate(cols, axis=1)
        valid = slot < cnt
        nbr = jnp.where(valid, nbrs, qloc) + f * NP
        nbr_ref[0, sl, :] = nbr
        val_ref[0, sl, :] = valid.astype(jnp.int32)


def _bq_call(x, y, z, sel, qx, qy, qz):
    frame_spec = pl.BlockSpec((1, 1, NP), lambda f: (f, 0, 0))
    q_spec = pl.BlockSpec((1, 1, NPOINT), lambda f: (f, 0, 0))
    out_spec = pl.BlockSpec((1, NPOINT, NSAMPLE), lambda f: (f, 0, 0))
    r3 = lambda a: a.reshape(F, 1, a.shape[-1])
    return pl.pallas_call(
        _bq_body,
        grid=(F,),
        in_specs=[frame_spec, frame_spec, frame_spec,
                  q_spec, q_spec, q_spec, q_spec],
        out_specs=(out_spec, out_spec),
        out_shape=(
            jax.ShapeDtypeStruct((F, NPOINT, NSAMPLE), jnp.int32),
            jax.ShapeDtypeStruct((F, NPOINT, NSAMPLE), jnp.int32),
        ),
    )(r3(x), r3(y), r3(z), r3(sel), r3(qx), r3(qy), r3(qz))


_GB = 4
_NGRP = 16


def _sc_gather(table, nbr_idx, q_idx):
    qns = nbr_idx.shape[0] * nbr_idx.shape[1]
    nq = q_idx.shape[0] * q_idx.shape[1]
    per_w = qns // _NW
    n_chunks = per_w // _CHUNK
    mesh = plsc.VectorSubcoreMesh(core_axis_name="c", subcore_axis_name="s")

    @functools.partial(
        pl.kernel,
        mesh=mesh,
        out_type=[
            jax.ShapeDtypeStruct((qns, D), jnp.float32),
            jax.ShapeDtypeStruct((nq, D), jnp.float32),
        ],
        scratch_types=[
            pltpu.VMEM((n_chunks, _CHUNK), jnp.int32),
            pltpu.VMEM((2, _CHUNK), jnp.int32),
            pltpu.VMEM((2 * _GB, _CHUNK, D), jnp.float32),
            pltpu.SemaphoreType.DMA,
            pltpu.SemaphoreType.DMA,
        ],
    )
    def k(table_hbm, nbr_hbm, q_hbm, outn_hbm, outq_hbm,
          idx_all, qidx, bufs, gsem, wsem):
        wid = lax.axis_index("s") * 2 + lax.axis_index("c")
        pltpu.sync_copy(nbr_hbm.at[pl.ds(wid * n_chunks, n_chunks)], idx_all)
        pltpu.sync_copy(q_hbm.at[pl.ds(wid * 2, 2)], qidx)
        out_base = wid * per_w

        def chunk_out(c):
            off = pl.multiple_of(out_base + c * _CHUNK, _CHUNK)
            return outn_hbm.at[pl.ds(off, _CHUNK)]

        def drain(sem):
            pltpu.make_async_copy(
                table_hbm.at[pl.ds(0, _CHUNK)], bufs.at[0], sem).wait()

        for b in range(2 * _GB):
            pltpu.async_copy(bufs.at[b], chunk_out(b), wsem)

        def group(g, carry):
            p = (g % 2) * _GB
            for b in range(_GB):
                drain(wsem)
            for b in range(_GB):
                c = g * _GB + b
                pltpu.async_copy(table_hbm.at[idx_all.at[c]],
                                 bufs.at[p + b], gsem)
            for b in range(_GB):
                drain(gsem)
            for b in range(_GB):
                c = g * _GB + b
                pltpu.async_copy(bufs.at[p + b], chunk_out(c), wsem)
            return carry

        lax.fori_loop(0, _NGRP, group, 0)
        for b in range(2 * _GB):
            drain(wsem)

        for t in range(2):
            pltpu.async_copy(table_hbm.at[qidx.at[t]], bufs.at[t], gsem)
        for t in range(2):
            drain(gsem)
        for t in range(2):
            off = pl.multiple_of(wid * 2 * _CHUNK + t * _CHUNK, _CHUNK)
            pltpu.async_copy(bufs.at[t], outq_hbm.at[pl.ds(off, _CHUNK)], wsem)
        for t in range(2):
            drain(wsem)

    return k(table, nbr_idx, q_idx)


_MQBLK = 256


def _mlp_body(g_ref, q_ref, val_ref, w1_ref, b1_ref, s1_ref, be1_ref,
              w2_ref, b2_ref, s2_ref, be2_ref, o_ref):
    g = g_ref[...]
    q = q_ref[...]
    col = lax.broadcasted_iota(jnp.int32, (_MQBLK, D), 1)
    qxyz = jnp.where(col < 3, q, 0.0)
    g3 = g.reshape(_MQBLK, NSAMPLE, D) - qxyz[:, None, :]
    a = g3.reshape(_MQBLK * NSAMPLE, D)
    y1 = jax.lax.dot_general(a, w1_ref[...], (((1,), (0,)), ((), ())),
                             preferred_element_type=jnp.float32)
    y1 = (y1 + b1_ref[...]) * s1_ref[...] + be1_ref[...]
    h1 = jnp.maximum(y1, 0.0)
    y2 = jax.lax.dot_general(h1, w2_ref[...], (((1,), (0,)), ((), ())),
                             preferred_element_type=jnp.float32)
    y2 = (y2 + b2_ref[...]) * s2_ref[...] + be2_ref[...]
    h2 = jnp.maximum(y2, 0.0)
    vrow = val_ref[...]
    hm = jnp.where(vrow > 0.5, h2, -jnp.inf)
    o_ref[...] = jnp.max(hm.reshape(_MQBLK, NSAMPLE, H2), axis=1).T


def _mlp_call(gn, gq, valid, w1t, b1, s1, be1, w2t, b2, s2, be2):
    nq = gq.shape[0]
    grid = (nq // _MQBLK,)
    full = lambda shape: pl.BlockSpec(shape, lambda i: tuple(0 for _ in shape))
    return pl.pallas_call(
        _mlp_body,
        grid=grid,
        in_specs=[
            pl.BlockSpec((_MQBLK * NSAMPLE, D), lambda i: (i, 0)),
            pl.BlockSpec((_MQBLK, D), lambda i: (i, 0)),
            pl.BlockSpec((_MQBLK * NSAMPLE, 1), lambda i: (i, 0)),
            full((D, H1)), full((1, H1)), full((1, H1)), full((1, H1)),
            full((H1, H2)), full((1, H2)), full((1, H2)), full((1, H2)),
        ],
        out_specs=pl.BlockSpec((H2, _MQBLK), lambda i: (0, i)),
        out_shape=jax.ShapeDtypeStruct((H2, nq), jnp.float32),
    )(gn, gq, valid, w1t, b1, s1, be1, w2t, b2, s2, be2)


def kernel(xyz, times, features, point2frameidx, frame2batchidx,
           W1, b1, g1, be1, W2, b2, g2, be2):
    p = xyz.reshape(F, NP, 3)
    x = p[:, :, 0]
    y = p[:, :, 1]
    z = p[:, :, 2]

    sel, qx, qy, qz = _fps_call(x, y, z)
    nbr, valid = _bq_call(x, y, z, sel, qx, qy, qz)

    offsets = (jnp.arange(F, dtype=jnp.int32) * NP)[:, None]
    inds = (sel + offsets).reshape(-1)

    pad = jnp.zeros((N, D - 3 - C), jnp.float32)
    table = jnp.concatenate([xyz, features, pad], axis=1)

    gn, gq = _sc_gather(table, nbr.reshape(-1, _CHUNK),
                        inds.reshape(-1, _CHUNK))

    s1 = (g1 / jnp.sqrt(1.0 + BN_EPS))[None, :]
    s2 = (g2 / jnp.sqrt(1.0 + BN_EPS))[None, :]
    w1t = jnp.zeros((D, H1), jnp.float32).at[:3 + C, :].set(W1.T)
    w2t = W2.T
    out = _mlp_call(gn, gq, valid.reshape(-1, 1).astype(jnp.float32),
                    w1t, b1[None, :], s1, be1[None, :],
                    w2t, b2[None, :], s2, be2[None, :])

    query_xyz = gq[:, :3][None]
    new_features = out[None]
    return query_xyz, new_features, inds


# speedup vs baseline: 1.1730x; 1.0135x over previous
import functools

import jax
import jax.numpy as jnp
from jax import lax
from jax.experimental import pallas as pl
from jax.experimental.pallas import tpu as pltpu
from jax.experimental.pallas import tpu_sc as plsc

N = 16384
F = 4
NP = N // F
C = 64
NPOINT = 1024
RADIUS2 = 0.2 * 0.2
NSAMPLE = 32
H1 = 64
H2 = 128
D = 128
BN_EPS = 1e-5

_NW = 32
_CHUNK = 64


def _fps_body(x_ref, y_ref, z_ref, sel_ref, qx_ref, qy_ref, qz_ref):
    x = x_ref[...]
    y = y_ref[...]
    z = z_ref[...]
    lane = lax.broadcasted_iota(jnp.int32, (F, NP), 1)
    qlane = lax.broadcasted_iota(jnp.int32, (F, NPOINT), 1)

    px = x[:, 0:1]
    py = y[:, 0:1]
    pz = z[:, 0:1]
    dx = x - px
    dy = y - py
    dz = z - pz
    mind = dx * dx + dy * dy + dz * dz

    lane128 = lax.broadcasted_iota(jnp.int32, (F, 128), 1)
    b0 = lane128 == 0
    bsel = jnp.zeros((F, 128), jnp.int32)
    bqx = jnp.where(b0, px, 0.0)
    bqy = jnp.where(b0, py, 0.0)
    bqz = jnp.where(b0, pz, 0.0)

    def body(i, carry):
        mind, bsel, bqx, bqy, bqz = carry
        m = jnp.max(mind, axis=1, keepdims=True)
        idx = jnp.min(jnp.where(mind == m, lane, NP), axis=1, keepdims=True)
        eq = lane == idx
        px = jnp.sum(jnp.where(eq, x, 0.0), axis=1, keepdims=True)
        py = jnp.sum(jnp.where(eq, y, 0.0), axis=1, keepdims=True)
        pz = jnp.sum(jnp.where(eq, z, 0.0), axis=1, keepdims=True)
        dx = x - px
        dy = y - py
        dz = z - pz
        d = dx * dx + dy * dy + dz * dz

        @pl.when(i % 128 == 0)
        def _flush():
            off = pl.multiple_of(i - 128, 128)
            sel_ref[:, pl.ds(off, 128)] = bsel
            qx_ref[:, pl.ds(off, 128)] = bqx
            qy_ref[:, pl.ds(off, 128)] = bqy
            qz_ref[:, pl.ds(off, 128)] = bqz

        here = lane128 == i % 128
        bsel = jnp.where(here, idx, bsel)
        bqx = jnp.where(here, px, bqx)
        bqy = jnp.where(here, py, bqy)
        bqz = jnp.where(here, pz, bqz)
        return jnp.minimum(mind, d), bsel, bqx, bqy, bqz

    _, bsel, bqx, bqy, bqz = lax.fori_loop(
        1, NPOINT, body, (mind, bsel, bqx, bqy, bqz))
    sel_ref[:, pl.ds(NPOINT - 128, 128)] = bsel
    qx_ref[:, pl.ds(NPOINT - 128, 128)] = bqx
    qy_ref[:, pl.ds(NPOINT - 128, 128)] = bqy
    qz_ref[:, pl.ds(NPOINT - 128, 128)] = bqz


def _fps_call(x, y, z):
    return pl.pallas_call(
        _fps_body,
        out_shape=(
            jax.ShapeDtypeStruct((F, NPOINT), jnp.int32),
            jax.ShapeDtypeStruct((F, NPOINT), jnp.float32),
            jax.ShapeDtypeStruct((F, NPOINT), jnp.float32),
            jax.ShapeDtypeStruct((F, NPOINT), jnp.float32),
        ),
    )(x, y, z)


_QBLK = 256


def _bq_body(x_ref, y_ref, z_ref, sel_ref, qx_ref, qy_ref, qz_ref,
             nbr_ref, val_ref):
    f = pl.program_id(0)
    x = x_ref[0]
    y = y_ref[0]
    z = z_ref[0]
    lane = lax.broadcasted_iota(jnp.int32, (_QBLK, NP), 1)
    slot = lax.broadcasted_iota(jnp.int32, (_QBLK, NSAMPLE), 1)

    for c in range(NPOINT // _QBLK):
        sl = pl.ds(c * _QBLK, _QBLK)
        qx = qx_ref[0, 0, sl][:, None]
        qy = qy_ref[0, 0, sl][:, None]
        qz = qz_ref[0, 0, sl][:, None]
        qloc = sel_ref[0, 0, sl][:, None]
        dx = qx - x
        dy = qy - y
        dz = qz - z
        d2 = dx * dx + dy * dy + dz * dz
        mask = d2 <= RADIUS2
        cnt = jnp.sum(mask.astype(jnp.int32), axis=1, keepdims=True)
        key0 = jnp.where(mask, lane, NP)
        m = jnp.min(key0, axis=1, keepdims=True)
        cols = [m]
        for _ in range(NSAMPLE - 1):
            m = jnp.min(jnp.where(lane > m, key0, NP), axis=1, keepdims=True)
            cols.append(m)
        nbrs = jnp.concatenate(cols, axis=1)
        valid = slot < cnt
        nbr = jnp.where(valid, nbrs, qloc) + f * NP
        nbr_ref[0, sl, :] = nbr
        val_ref[0, sl, :] = valid.astype(jnp.int32)


def _bq_call(x, y, z, sel, qx, qy, qz):
    frame_spec = pl.BlockSpec((1, 1, NP), lambda f: (f, 0, 0))
    q_spec = pl.BlockSpec((1, 1, NPOINT), lambda f: (f, 0, 0))
    out_spec = pl.BlockSpec((1, NPOINT, NSAMPLE), lambda f: (f, 0, 0))
    r3 = lambda a: a.reshape(F, 1, a.shape[-1])
    return pl.pallas_call(
        _bq_body,
        grid=(F,),
        in_specs=[frame_spec, frame_spec, frame_spec,
                  q_spec, q_spec, q_spec, q_spec],
        out_specs=(out_spec, out_spec),
        out_shape=(
            jax.ShapeDtypeStruct((F, NPOINT, NSAMPLE), jnp.int32),
            jax.ShapeDtypeStruct((F, NPOINT, NSAMPLE), jnp.int32),
        ),
    )(r3(x), r3(y), r3(z), r3(sel), r3(qx), r3(qy), r3(qz))


_GB = 4
_NGRP = 16


def _sc_gather(table, nbr_idx, q_idx):
    qns = nbr_idx.shape[0] * nbr_idx.shape[1]
    nq = q_idx.shape[0] * q_idx.shape[1]
    per_w = qns // _NW
    n_chunks = per_w // _CHUNK
    mesh = plsc.VectorSubcoreMesh(core_axis_name="c", subcore_axis_name="s")

    @functools.partial(
        pl.kernel,
        mesh=mesh,
        out_type=[
            jax.ShapeDtypeStruct((qns, D), jnp.float32),
            jax.ShapeDtypeStruct((nq, D), jnp.float32),
        ],
        scratch_types=[
            pltpu.VMEM((n_chunks, _CHUNK), jnp.int32),
            pltpu.VMEM((2, _CHUNK), jnp.int32),
            pltpu.VMEM((2 * _GB, _CHUNK, D), jnp.float32),
            pltpu.SemaphoreType.DMA,
            pltpu.SemaphoreType.DMA,
        ],
    )
    def k(table_hbm, nbr_hbm, q_hbm, outn_hbm, outq_hbm,
          idx_all, qidx, bufs, gsem, wsem):
        wid = lax.axis_index("s") * 2 + lax.axis_index("c")
        pltpu.sync_copy(nbr_hbm.at[pl.ds(wid * n_chunks, n_chunks)], idx_all)
        pltpu.sync_copy(q_hbm.at[pl.ds(wid * 2, 2)], qidx)
        out_base = wid * per_w

        def chunk_out(c):
            off = pl.multiple_of(out_base + c * _CHUNK, _CHUNK)
            return outn_hbm.at[pl.ds(off, _CHUNK)]

        def drain(sem):
            pltpu.make_async_copy(
                table_hbm.at[pl.ds(0, _CHUNK)], bufs.at[0], sem).wait()

        for b in range(2 * _GB):
            pltpu.async_copy(bufs.at[b], chunk_out(b), wsem)

        def group(g, carry):
            p = (g % 2) * _GB
            for b in range(_GB):
                drain(wsem)
            for b in range(_GB):
                c = g * _GB + b
                pltpu.async_copy(table_hbm.at[idx_all.at[c]],
                                 bufs.at[p + b], gsem)
            for b in range(_GB):
                drain(gsem)
            for b in range(_GB):
                c = g * _GB + b
                pltpu.async_copy(bufs.at[p + b], chunk_out(c), wsem)
            return carry

        lax.fori_loop(0, _NGRP, group, 0)
        for b in range(2 * _GB):
            drain(wsem)

        for t in range(2):
            pltpu.async_copy(table_hbm.at[qidx.at[t]], bufs.at[t], gsem)
        for t in range(2):
            drain(gsem)
        for t in range(2):
            off = pl.multiple_of(wid * 2 * _CHUNK + t * _CHUNK, _CHUNK)
            pltpu.async_copy(bufs.at[t], outq_hbm.at[pl.ds(off, _CHUNK)], wsem)
        for t in range(2):
            drain(wsem)

    return k(table, nbr_idx, q_idx)


_MQBLK = 256


def _mlp_body(g_ref, q_ref, val_ref, w1_ref, b1_ref, s1_ref, be1_ref,
              w2_ref, b2_ref, s2_ref, be2_ref, o_ref):
    g = g_ref[...]
    q = q_ref[...]
    col = lax.broadcasted_iota(jnp.int32, (_MQBLK, D), 1)
    qxyz = jnp.where(col < 3, q, 0.0)
    g3 = g.reshape(_MQBLK, NSAMPLE, D) - qxyz[:, None, :]
    a = g3.reshape(_MQBLK * NSAMPLE, D)
    y1 = jax.lax.dot_general(a, w1_ref[...], (((1,), (0,)), ((), ())),
                             preferred_element_type=jnp.float32)
    y1 = (y1 + b1_ref[...]) * s1_ref[...] + be1_ref[...]
    h1 = jnp.maximum(y1, 0.0)
    y2 = jax.lax.dot_general(h1, w2_ref[...], (((1,), (0,)), ((), ())),
                             preferred_element_type=jnp.float32)
    y2 = (y2 + b2_ref[...]) * s2_ref[...] + be2_ref[...]
    h2 = jnp.maximum(y2, 0.0)
    vrow = val_ref[...]
    hm = jnp.where(vrow > 0.5, h2, -jnp.inf)
    o_ref[...] = jnp.max(hm.reshape(_MQBLK, NSAMPLE, H2), axis=1).T


def _mlp_call(gn, gq, valid, w1t, b1, s1, be1, w2t, b2, s2, be2):
    nq = gq.shape[0]
    grid = (nq // _MQBLK,)
    full = lambda shape: pl.BlockSpec(shape, lambda i: tuple(0 for _ in shape))
    return pl.pallas_call(
        _mlp_body,
        grid=grid,
        in_specs=[
            pl.BlockSpec((_MQBLK * NSAMPLE, D), lambda i: (i, 0)),
            pl.BlockSpec((_MQBLK, D), lambda i: (i, 0)),
            pl.BlockSpec((_MQBLK * NSAMPLE, 1), lambda i: (i, 0)),
            full((D, H1)), full((1, H1)), full((1, H1)), full((1, H1)),
            full((H1, H2)), full((1, H2)), full((1, H2)), full((1, H2)),
        ],
        out_specs=pl.BlockSpec((H2, _MQBLK), lambda i: (0, i)),
        out_shape=jax.ShapeDtypeStruct((H2, nq), jnp.float32),
    )(gn, gq, valid, w1t, b1, s1, be1, w2t, b2, s2, be2)


def kernel(xyz, times, features, point2frameidx, frame2batchidx,
           W1, b1, g1, be1, W2, b2, g2, be2):
    p = xyz.reshape(F, NP, 3)
    x = p[:, :, 0]
    y = p[:, :, 1]
    z = p[:, :, 2]

    sel, qx, qy, qz = _fps_call(x, y, z)
    nbr, valid = _bq_call(x, y, z, sel, qx, qy, qz)

    offsets = (jnp.arange(F, dtype=jnp.int32) * NP)[:, None]
    inds = (sel + offsets).reshape(-1)

    pad = jnp.zeros((N, D - 3 - C), jnp.float32)
    table = jnp.concatenate([xyz, features, pad], axis=1)

    gn, gq = _sc_gather(table, nbr.reshape(-1, _CHUNK),
                        inds.reshape(-1, _CHUNK))

    s1 = (g1 / jnp.sqrt(1.0 + BN_EPS))[None, :]
    s2 = (g2 / jnp.sqrt(1.0 + BN_EPS))[None, :]
    w1t = jnp.zeros((D, H1), jnp.float32).at[:3 + C, :].set(W1.T)
    w2t = W2.T
    out = _mlp_call(gn, gq, valid.reshape(-1, 1).astype(jnp.float32),
                    w1t, b1[None, :], s1, be1[None, :],
                    w2t, b2[None, :], s2, be2[None, :])

    query_xyz = gq[:, :3][None]
    new_features = out[None]
    return query_xyz, new_features, inds
